# baseline (speedup 1.0000x reference)
import functools

import jax
import jax.numpy as jnp
from jax import lax
from jax.experimental import pallas as pl
from jax.experimental.pallas import tpu as pltpu
from jax.experimental.pallas import tpu_sc as plsc

BATCH = 16384
FEAT = 2048
LANES = 16
NUM_CORES = 2
NUM_SUBCORES = 16
NW = NUM_CORES * NUM_SUBCORES
ROWS_PER_W = BATCH // NW
R = 16
NCHUNKS = ROWS_PER_W // R
JBLKS = FEAT // LANES


@functools.partial(
    pl.kernel,
    mesh=plsc.VectorSubcoreMesh(core_axis_name="c", subcore_axis_name="s"),
    out_type=jax.ShapeDtypeStruct((BATCH * FEAT,), jnp.float32),
    compiler_params=pltpu.CompilerParams(needs_layout_passes=False),
    scratch_types=[
        pltpu.VMEM((FEAT,), jnp.int32),
        pltpu.VMEM((R * FEAT,), jnp.float32),
        pltpu.VMEM((R * FEAT,), jnp.float32),
    ],
)
def _shuffle(in_hbm, perm_hbm, out_hbm, perm_v, in_v, out_v):
    wid = lax.axis_index("s") * NUM_CORES + lax.axis_index("c")
    base = wid * ROWS_PER_W * FEAT
    pltpu.sync_copy(perm_hbm, perm_v)

    def chunk_body(c, carry):
        off = base + c * (R * FEAT)
        pltpu.sync_copy(in_hbm.at[pl.ds(off, R * FEAT)], in_v)

        def j_body(j, carry2):
            idx = perm_v[pl.ds(j * LANES, LANES)]
            for r in range(R):
                vals = plsc.load_gather(in_v, [idx + (r * FEAT)])
                out_v[pl.ds(r * FEAT + j * LANES, LANES)] = vals
            return carry2

        lax.fori_loop(0, JBLKS, j_body, 0)
        pltpu.sync_copy(out_v, out_hbm.at[pl.ds(off, R * FEAT)])
        return carry

    lax.fori_loop(0, NCHUNKS, chunk_body, 0)


def kernel(inputs, perm):
    out_flat = _shuffle(inputs.reshape(-1), perm)
    out = out_flat.reshape(BATCH, FEAT)
    logdet = jnp.zeros((inputs.shape[0], 1), dtype=inputs.dtype)
    return (out, logdet)


# speedup vs baseline: 1.8929x; 1.8929x over previous
import functools

import jax
import jax.numpy as jnp
from jax import lax
from jax.experimental import pallas as pl
from jax.experimental.pallas import tpu as pltpu
from jax.experimental.pallas import tpu_sc as plsc

BATCH = 16384
FEAT = 2048
LANES = 16
NUM_CORES = 2
NUM_SUBCORES = 16
NW = NUM_CORES * NUM_SUBCORES
ROWS_PER_W = BATCH // NW
R = 8
CH = R * FEAT
NCHUNKS = ROWS_PER_W // R
JBLKS = FEAT // LANES


@functools.partial(
    pl.kernel,
    mesh=plsc.VectorSubcoreMesh(core_axis_name="c", subcore_axis_name="s"),
    out_type=jax.ShapeDtypeStruct((BATCH * FEAT,), jnp.float32),
    compiler_params=pltpu.CompilerParams(needs_layout_passes=False),
    scratch_types=[
        pltpu.VMEM((FEAT,), jnp.int32),
        pltpu.VMEM((CH,), jnp.float32),
        pltpu.VMEM((CH,), jnp.float32),
        pltpu.VMEM((CH,), jnp.float32),
        pltpu.VMEM((CH,), jnp.float32),
        pltpu.SemaphoreType.DMA,
        pltpu.SemaphoreType.DMA,
        pltpu.SemaphoreType.DMA,
        pltpu.SemaphoreType.DMA,
    ],
)
def _shuffle(in_hbm, perm_hbm, out_hbm, perm_v, in_v0, in_v1, out_v0,
             out_v1, si0, si1, so0, so1):
    wid = lax.axis_index("s") * NUM_CORES + lax.axis_index("c")
    base = wid * ROWS_PER_W * FEAT
    pltpu.sync_copy(perm_hbm, perm_v)

    in_bufs = (in_v0, in_v1)
    out_bufs = (out_v0, out_v1)
    sem_in = (si0, si1)
    sem_out = (so0, so1)

    def start_in(c, b):
        pltpu.async_copy(in_hbm.at[pl.ds(base + c * CH, CH)], in_bufs[b],
                         sem_in[b])

    def wait_in(b):
        pltpu.make_async_copy(in_hbm.at[pl.ds(0, CH)], in_bufs[b],
                              sem_in[b]).wait()

    def start_out(c, b):
        pltpu.async_copy(out_bufs[b], out_hbm.at[pl.ds(base + c * CH, CH)],
                         sem_out[b])

    def wait_out(b):
        pltpu.make_async_copy(out_bufs[b], out_hbm.at[pl.ds(0, CH)],
                              sem_out[b]).wait()

    def compute(b):
        in_b, out_b = in_bufs[b], out_bufs[b]

        @plsc.parallel_loop(0, JBLKS, unroll=2)
        def _(j):
            col = j * LANES
            idx = perm_v[pl.ds(col, LANES)]
            for r in range(R):
                out_b[pl.ds(r * FEAT + col, LANES)] = plsc.load_gather(
                    in_b, [idx + (r * FEAT)]
                )

    start_in(0, 0)
    start_in(1, 1)
    for b in (0, 1):
        wait_in(b)
        compute(b)
        start_out(b, b)
        start_in(b + 2, b)

    @pl.loop(2, NCHUNKS - 2, step=2)
    def _(c0):
        for b in (0, 1):
            c = c0 + b
            wait_in(b)
            wait_out(b)
            compute(b)
            start_out(c, b)
            start_in(c + 2, b)

    for b in (0, 1):
        c = NCHUNKS - 2 + b
        wait_in(b)
        wait_out(b)
        compute(b)
        start_out(c, b)
    wait_out(0)
    wait_out(1)


def kernel(inputs, perm):
    out_flat = _shuffle(inputs.reshape(-1), perm)
    out = out_flat.reshape(BATCH, FEAT)
    logdet = jnp.zeros((inputs.shape[0], 1), dtype=inputs.dtype)
    return (out, logdet)


# speedup vs baseline: 5.4548x; 2.8818x over previous
import functools

import jax
import jax.numpy as jnp
from jax import lax
from jax.experimental import pallas as pl
from jax.experimental.pallas import tpu as pltpu
from jax.experimental.pallas import tpu_sc as plsc

BATCH = 16384
FEAT = 2048
LANES = 16
NUM_CORES = 2
NUM_SUBCORES = 16
NW = NUM_CORES * NUM_SUBCORES
ROWS_PER_W = BATCH // NW
R = 8
CH = R * FEAT
NCHUNKS = ROWS_PER_W // R
JBLKS = FEAT // LANES


@functools.partial(
    pl.kernel,
    mesh=plsc.VectorSubcoreMesh(core_axis_name="c", subcore_axis_name="s"),
    out_type=jax.ShapeDtypeStruct((BATCH, FEAT), jnp.float32),
    compiler_params=pltpu.CompilerParams(
        needs_layout_passes=False, use_tc_tiling_on_sc=True
    ),
    scratch_types=[
        pltpu.VMEM((FEAT,), jnp.int32),
        pltpu.VMEM((R, FEAT), jnp.float32),
        pltpu.VMEM((R, FEAT), jnp.float32),
        pltpu.VMEM((R, FEAT), jnp.float32),
        pltpu.VMEM((R, FEAT), jnp.float32),
        pltpu.SemaphoreType.DMA,
        pltpu.SemaphoreType.DMA,
        pltpu.SemaphoreType.DMA,
        pltpu.SemaphoreType.DMA,
    ],
)
def _shuffle(in_hbm, perm_hbm, out_hbm, perm_v, in_v0, in_v1, out_v0,
             out_v1, si0, si1, so0, so1):
    wid = lax.axis_index("s") * NUM_CORES + lax.axis_index("c")
    row_base = wid * ROWS_PER_W
    pltpu.sync_copy(perm_hbm, perm_v)

    in_bufs = (in_v0, in_v1)
    out_bufs = (out_v0, out_v1)
    sem_in = (si0, si1)
    sem_out = (so0, so1)

    def start_in(c, b):
        pltpu.async_copy(in_hbm.at[pl.ds(row_base + c * R, R)], in_bufs[b],
                         sem_in[b])

    def wait_in(b):
        pltpu.make_async_copy(in_hbm.at[pl.ds(0, R)], in_bufs[b],
                              sem_in[b]).wait()

    def start_out(c, b):
        pltpu.async_copy(out_bufs[b], out_hbm.at[pl.ds(row_base + c * R, R)],
                         sem_out[b])

    def wait_out(b):
        pltpu.make_async_copy(out_bufs[b], out_hbm.at[pl.ds(0, R)],
                              sem_out[b]).wait()

    def compute(b):
        in_b, out_b = in_bufs[b], out_bufs[b]

        @plsc.parallel_loop(0, JBLKS, unroll=2)
        def _(j):
            col = j * LANES
            idx = perm_v[pl.ds(col, LANES)]
            for r in range(R):
                row_idx = jnp.full((LANES,), r, jnp.int32)
                out_b[r, pl.ds(col, LANES)] = plsc.load_gather(
                    in_b, [row_idx, idx]
                )

    start_in(0, 0)
    start_in(1, 1)
    for b in (0, 1):
        wait_in(b)
        compute(b)
        start_out(b, b)
        start_in(b + 2, b)

    @pl.loop(2, NCHUNKS - 2, step=2)
    def _(c0):
        for b in (0, 1):
            c = c0 + b
            wait_in(b)
            wait_out(b)
            compute(b)
            start_out(c, b)
            start_in(c + 2, b)

    for b in (0, 1):
        c = NCHUNKS - 2 + b
        wait_in(b)
        wait_out(b)
        compute(b)
        start_out(c, b)
    wait_out(0)
    wait_out(1)


def kernel(inputs, perm):
    out = _shuffle(inputs, perm)
    logdet = jnp.zeros((inputs.shape[0], 1), dtype=inputs.dtype)
    return (out, logdet)


# speedup vs baseline: 5.4630x; 1.0015x over previous
import functools

import jax
import jax.numpy as jnp
from jax import lax
from jax.experimental import pallas as pl
from jax.experimental.pallas import tpu as pltpu
from jax.experimental.pallas import tpu_sc as plsc

BATCH = 16384
FEAT = 2048
LANES = 16
NUM_CORES = 2
NUM_SUBCORES = 16
NW = NUM_CORES * NUM_SUBCORES
ROWS_PER_W = BATCH // NW
R = 8
CH = R * FEAT
NCHUNKS = ROWS_PER_W // R
JBLKS = FEAT // LANES


@functools.partial(
    pl.kernel,
    mesh=plsc.VectorSubcoreMesh(core_axis_name="c", subcore_axis_name="s"),
    out_type=jax.ShapeDtypeStruct((BATCH, FEAT), jnp.float32),
    compiler_params=pltpu.CompilerParams(
        needs_layout_passes=False, use_tc_tiling_on_sc=True
    ),
    scratch_types=[
        pltpu.VMEM((FEAT,), jnp.int32),
        pltpu.VMEM((R, FEAT), jnp.float32),
        pltpu.VMEM((R, FEAT), jnp.float32),
        pltpu.VMEM((R, FEAT), jnp.float32),
        pltpu.VMEM((R, FEAT), jnp.float32),
        pltpu.SemaphoreType.DMA,
        pltpu.SemaphoreType.DMA,
        pltpu.SemaphoreType.DMA,
        pltpu.SemaphoreType.DMA,
    ],
)
def _shuffle(in_hbm, perm_hbm, out_hbm, perm_v, in_v0, in_v1, out_v0,
             out_v1, si0, si1, so0, so1):
    wid = lax.axis_index("s") * NUM_CORES + lax.axis_index("c")
    row_base = wid * ROWS_PER_W
    pltpu.sync_copy(perm_hbm, perm_v)

    in_bufs = (in_v0, in_v1)
    out_bufs = (out_v0, out_v1)
    sem_in = (si0, si1)
    sem_out = (so0, so1)

    def start_in(c, b):
        pltpu.async_copy(in_hbm.at[pl.ds(row_base + c * R, R)], in_bufs[b],
                         sem_in[b])

    def wait_in(b):
        pltpu.make_async_copy(in_hbm.at[pl.ds(0, R)], in_bufs[b],
                              sem_in[b]).wait()

    def start_out(c, b):
        pltpu.async_copy(out_bufs[b], out_hbm.at[pl.ds(row_base + c * R, R)],
                         sem_out[b])

    def wait_out(b):
        pltpu.make_async_copy(out_bufs[b], out_hbm.at[pl.ds(0, R)],
                              sem_out[b]).wait()

    def compute(b):
        in_b, out_b = in_bufs[b], out_bufs[b]

        @plsc.parallel_loop(0, JBLKS, unroll=4)
        def _(j):
            col = j * LANES
            idx = perm_v[pl.ds(col, LANES)]
            for r in range(R):
                row_idx = jnp.full((LANES,), r, jnp.int32)
                out_b[r, pl.ds(col, LANES)] = plsc.load_gather(
                    in_b, [row_idx, idx]
                )

    start_in(0, 0)
    start_in(1, 1)
    for b in (0, 1):
        wait_in(b)
        compute(b)
        start_out(b, b)
        start_in(b + 2, b)

    @pl.loop(2, NCHUNKS - 2, step=2)
    def _(c0):
        for b in (0, 1):
            c = c0 + b
            wait_in(b)
            wait_out(b)
            compute(b)
            start_out(c, b)
            start_in(c + 2, b)

    for b in (0, 1):
        c = NCHUNKS - 2 + b
        wait_in(b)
        wait_out(b)
        compute(b)
        start_out(c, b)
    wait_out(0)
    wait_out(1)


def kernel(inputs, perm):
    out = _shuffle(inputs, perm)
    logdet = jnp.zeros((inputs.shape[0], 1), dtype=inputs.dtype)
    return (out, logdet)


# speedup vs baseline: 5.6060x; 1.0262x over previous
import functools

import jax
import jax.numpy as jnp
from jax import lax
from jax.experimental import pallas as pl
from jax.experimental.pallas import tpu as pltpu
from jax.experimental.pallas import tpu_sc as plsc

BATCH = 16384
FEAT = 2048
LANES = 16
NUM_CORES = 2
NUM_SUBCORES = 16
NW = NUM_CORES * NUM_SUBCORES
ROWS_PER_W = BATCH // NW
R = 4
NBUF = 4
NCHUNKS = ROWS_PER_W // R
JBLKS = FEAT // LANES


@functools.partial(
    pl.kernel,
    mesh=plsc.VectorSubcoreMesh(core_axis_name="c", subcore_axis_name="s"),
    out_type=jax.ShapeDtypeStruct((BATCH, FEAT), jnp.float32),
    compiler_params=pltpu.CompilerParams(
        needs_layout_passes=False, use_tc_tiling_on_sc=True
    ),
    scratch_types=(
        [pltpu.VMEM((FEAT,), jnp.int32)]
        + [pltpu.VMEM((R, FEAT), jnp.float32) for _ in range(2 * NBUF)]
        + [pltpu.SemaphoreType.DMA for _ in range(2 * NBUF)]
    ),
)
def _shuffle(in_hbm, perm_hbm, out_hbm, perm_v, *bufs_and_sems):
    in_bufs = bufs_and_sems[0:NBUF]
    out_bufs = bufs_and_sems[NBUF:2 * NBUF]
    sem_in = bufs_and_sems[2 * NBUF:3 * NBUF]
    sem_out = bufs_and_sems[3 * NBUF:4 * NBUF]

    wid = lax.axis_index("s") * NUM_CORES + lax.axis_index("c")
    row_base = wid * ROWS_PER_W
    pltpu.sync_copy(perm_hbm, perm_v)

    def start_in(c, b):
        pltpu.async_copy(in_hbm.at[pl.ds(row_base + c * R, R)], in_bufs[b],
                         sem_in[b])

    def wait_in(b):
        pltpu.make_async_copy(in_hbm.at[pl.ds(0, R)], in_bufs[b],
                              sem_in[b]).wait()

    def start_out(c, b):
        pltpu.async_copy(out_bufs[b], out_hbm.at[pl.ds(row_base + c * R, R)],
                         sem_out[b])

    def wait_out(b):
        pltpu.make_async_copy(out_bufs[b], out_hbm.at[pl.ds(0, R)],
                              sem_out[b]).wait()

    def compute(b):
        in_b, out_b = in_bufs[b], out_bufs[b]

        @plsc.parallel_loop(0, JBLKS, unroll=4)
        def _(j):
            col = j * LANES
            idx = perm_v[pl.ds(col, LANES)]
            for r in range(R):
                row_idx = jnp.full((LANES,), r, jnp.int32)
                out_b[r, pl.ds(col, LANES)] = plsc.load_gather(
                    in_b, [row_idx, idx]
                )

    for b in range(NBUF):
        start_in(b, b)
    for b in range(NBUF):
        wait_in(b)
        compute(b)
        start_out(b, b)
        start_in(b + NBUF, b)

    @pl.loop(NBUF, NCHUNKS - NBUF, step=NBUF)
    def _(c0):
        for b in range(NBUF):
            c = c0 + b
            wait_in(b)
            wait_out(b)
            compute(b)
            start_out(c, b)
            start_in(c + NBUF, b)

    for b in range(NBUF):
        c = NCHUNKS - NBUF + b
        wait_in(b)
        wait_out(b)
        compute(b)
        start_out(c, b)
    for b in range(NBUF):
        wait_out(b)


def kernel(inputs, perm):
    out = _shuffle(inputs, perm)
    logdet = jnp.zeros((inputs.shape[0], 1), dtype=inputs.dtype)
    return (out, logdet)


# speedup vs baseline: 5.6118x; 1.0010x over previous
import functools

import jax
import jax.numpy as jnp
from jax import lax
from jax.experimental import pallas as pl
from jax.experimental.pallas import tpu as pltpu
from jax.experimental.pallas import tpu_sc as plsc

BATCH = 16384
FEAT = 2048
LANES = 16
NUM_CORES = 2
NUM_SUBCORES = 16
NW = NUM_CORES * NUM_SUBCORES
ROWS_PER_W = BATCH // NW
R = 4
NBUF = 4
NCHUNKS = ROWS_PER_W // R
JBLKS = FEAT // LANES


@functools.partial(
    pl.kernel,
    mesh=plsc.VectorSubcoreMesh(core_axis_name="c", subcore_axis_name="s"),
    out_type=jax.ShapeDtypeStruct((BATCH, FEAT), jnp.float32),
    compiler_params=pltpu.CompilerParams(
        needs_layout_passes=False, use_tc_tiling_on_sc=True
    ),
    scratch_types=(
        [pltpu.VMEM((FEAT,), jnp.int32)]
        + [pltpu.VMEM((R, FEAT), jnp.float32) for _ in range(2 * NBUF)]
        + [pltpu.SemaphoreType.DMA for _ in range(2 * NBUF)]
    ),
)
def _shuffle(in_hbm, perm_hbm, out_hbm, perm_v, *bufs_and_sems):
    in_bufs = bufs_and_sems[0:NBUF]
    out_bufs = bufs_and_sems[NBUF:2 * NBUF]
    sem_in = bufs_and_sems[2 * NBUF:3 * NBUF]
    sem_out = bufs_and_sems[3 * NBUF:4 * NBUF]

    wid = lax.axis_index("s") * NUM_CORES + lax.axis_index("c")
    row_base = wid * ROWS_PER_W

    def start_in(c, b):
        pltpu.async_copy(in_hbm.at[pl.ds(row_base + c * R, R)], in_bufs[b],
                         sem_in[b])

    def wait_in(b):
        pltpu.make_async_copy(in_hbm.at[pl.ds(0, R)], in_bufs[b],
                              sem_in[b]).wait()

    def start_out(c, b):
        pltpu.async_copy(out_bufs[b], out_hbm.at[pl.ds(row_base + c * R, R)],
                         sem_out[b])

    def wait_out(b):
        pltpu.make_async_copy(out_bufs[b], out_hbm.at[pl.ds(0, R)],
                              sem_out[b]).wait()

    def compute(b):
        in_b, out_b = in_bufs[b], out_bufs[b]

        @plsc.parallel_loop(0, JBLKS, unroll=4)
        def _(j):
            col = j * LANES
            idx = perm_v[pl.ds(col, LANES)]
            for r in range(R):
                row_idx = jnp.full((LANES,), r, jnp.int32)
                out_b[r, pl.ds(col, LANES)] = plsc.load_gather(
                    in_b, [row_idx, idx]
                )

    for b in range(NBUF):
        start_in(b, b)
    pltpu.sync_copy(perm_hbm, perm_v)
    for b in range(NBUF):
        wait_in(b)
        compute(b)
        start_out(b, b)
        start_in(b + NBUF, b)

    @pl.loop(NBUF, NCHUNKS - NBUF, step=NBUF)
    def _(c0):
        for b in range(NBUF):
            c = c0 + b
            wait_in(b)
            wait_out(b)
            compute(b)
            start_out(c, b)
            start_in(c + NBUF, b)

    for b in range(NBUF):
        c = NCHUNKS - NBUF + b
        wait_in(b)
        wait_out(b)
        compute(b)
        start_out(c, b)
    for b in range(NBUF):
        wait_out(b)


def kernel(inputs, perm):
    out = _shuffle(inputs, perm)
    logdet = jnp.zeros((inputs.shape[0], 1), dtype=inputs.dtype)
    return (out, logdet)


# speedup vs baseline: 5.6122x; 1.0001x over previous
import functools

import jax
import jax.numpy as jnp
from jax import lax
from jax.experimental import pallas as pl
from jax.experimental.pallas import tpu as pltpu
from jax.experimental.pallas import tpu_sc as plsc

BATCH = 16384
FEAT = 2048
LANES = 16
NUM_CORES = 2
NUM_SUBCORES = 16
NW = NUM_CORES * NUM_SUBCORES
ROWS_PER_W = BATCH // NW
R = 4
NBUF = 4
NCHUNKS = ROWS_PER_W // R
JBLKS = FEAT // LANES


@functools.partial(
    pl.kernel,
    mesh=plsc.VectorSubcoreMesh(core_axis_name="c", subcore_axis_name="s"),
    out_type=jax.ShapeDtypeStruct((BATCH, FEAT), jnp.float32),
    compiler_params=pltpu.CompilerParams(
        needs_layout_passes=False, use_tc_tiling_on_sc=True, skip_device_barrier=True
    ),
    scratch_types=(
        [pltpu.VMEM((FEAT,), jnp.int32)]
        + [pltpu.VMEM((R, FEAT), jnp.float32) for _ in range(2 * NBUF)]
        + [pltpu.SemaphoreType.DMA for _ in range(2 * NBUF)]
    ),
)
def _shuffle(in_hbm, perm_hbm, out_hbm, perm_v, *bufs_and_sems):
    in_bufs = bufs_and_sems[0:NBUF]
    out_bufs = bufs_and_sems[NBUF:2 * NBUF]
    sem_in = bufs_and_sems[2 * NBUF:3 * NBUF]
    sem_out = bufs_and_sems[3 * NBUF:4 * NBUF]

    wid = lax.axis_index("s") * NUM_CORES + lax.axis_index("c")
    row_base = wid * ROWS_PER_W

    def start_in(c, b):
        pltpu.async_copy(in_hbm.at[pl.ds(row_base + c * R, R)], in_bufs[b],
                         sem_in[b])

    def wait_in(b):
        pltpu.make_async_copy(in_hbm.at[pl.ds(0, R)], in_bufs[b],
                              sem_in[b]).wait()

    def start_out(c, b):
        pltpu.async_copy(out_bufs[b], out_hbm.at[pl.ds(row_base + c * R, R)],
                         sem_out[b])

    def wait_out(b):
        pltpu.make_async_copy(out_bufs[b], out_hbm.at[pl.ds(0, R)],
                              sem_out[b]).wait()

    def compute(b):
        in_b, out_b = in_bufs[b], out_bufs[b]

        @plsc.parallel_loop(0, JBLKS, unroll=4)
        def _(j):
            col = j * LANES
            idx = perm_v[pl.ds(col, LANES)]
            for r in range(R):
                row_idx = jnp.full((LANES,), r, jnp.int32)
                out_b[r, pl.ds(col, LANES)] = plsc.load_gather(
                    in_b, [row_idx, idx]
                )

    for b in range(NBUF):
        start_in(b, b)
    pltpu.sync_copy(perm_hbm, perm_v)
    for b in range(NBUF):
        wait_in(b)
        compute(b)
        start_out(b, b)
        start_in(b + NBUF, b)

    @pl.loop(NBUF, NCHUNKS - NBUF, step=NBUF)
    def _(c0):
        for b in range(NBUF):
            c = c0 + b
            wait_in(b)
            wait_out(b)
            compute(b)
            start_out(c, b)
            start_in(c + NBUF, b)

    for b in range(NBUF):
        c = NCHUNKS - NBUF + b
        wait_in(b)
        wait_out(b)
        compute(b)
        start_out(c, b)
    for b in range(NBUF):
        wait_out(b)


def kernel(inputs, perm):
    out = _shuffle(inputs, perm)
    logdet = jnp.zeros((inputs.shape[0], 1), dtype=inputs.dtype)
    return (out, logdet)
